# baseline (speedup 1.0000x reference)
import functools

import jax
import jax.numpy as jnp
from jax.experimental import pallas as pl
from jax.experimental.pallas import tpu as pltpu


def _gnn_block(out_dim, a_ud_ref, a_lr_ref, xf_ref, wcat_ref, wlin_ref,
               ball_ref, out_ref, y_ref, base_ref, acc_ref):
    j = pl.program_id(0)

    @pl.when(j == 0)
    def _():
        xf = xf_ref[...]
        y_ref[...] = jnp.dot(xf, wcat_ref[...],
                             preferred_element_type=jnp.float32)
        base_ref[...] = (jnp.dot(xf, wlin_ref[...],
                                 preferred_element_type=jnp.float32)
                         + ball_ref[...])

    y = y_ref[...]
    bm = out_ref.shape[0]

    @pl.when(j % 2 == 0)
    def _():
        acc_ref[...] = jnp.dot(a_ud_ref[...], y[:, :out_dim],
                               preferred_element_type=jnp.float32)

    @pl.when(j % 2 == 1)
    def _():
        acc = acc_ref[...] + jnp.dot(a_lr_ref[...], y[:, out_dim:],
                                     preferred_element_type=jnp.float32)
        i = j // 2
        out_ref[...] = jnp.maximum(acc + base_ref[pl.ds(i * bm, bm), :], 0.0)


def kernel(x, mask, a_ud, a_lr, W_lin, b_lin, W_ud, b_ud, W_lr, b_lr):
    num_sent, sent_len, hidden = x.shape
    n = num_sent * sent_len
    out_dim = W_lin.shape[0]
    xf = x.reshape(n, hidden)
    wcat = jnp.concatenate([W_ud.T, W_lr.T], axis=1)
    wlin = W_lin.T
    ball = (b_lin + b_ud + b_lr).reshape(1, out_dim)

    bm = 256
    grid = (2 * (n // bm),)
    h = pl.pallas_call(
        functools.partial(_gnn_block, out_dim),
        grid=grid,
        in_specs=[
            pl.BlockSpec((bm, n), lambda j: (j // 2, 0)),
            pl.BlockSpec((bm, n), lambda j: (jnp.maximum(j - 1, 0) // 2, 0)),
            pl.BlockSpec((n, hidden), lambda j: (0, 0)),
            pl.BlockSpec((hidden, 2 * out_dim), lambda j: (0, 0)),
            pl.BlockSpec((hidden, out_dim), lambda j: (0, 0)),
            pl.BlockSpec((1, out_dim), lambda j: (0, 0)),
        ],
        out_specs=pl.BlockSpec((bm, out_dim), lambda j: (j // 2, 0)),
        out_shape=jax.ShapeDtypeStruct((n, out_dim), jnp.float32),
        scratch_shapes=[
            pltpu.VMEM((n, 2 * out_dim), jnp.float32),
            pltpu.VMEM((n, out_dim), jnp.float32),
            pltpu.VMEM((bm, out_dim), jnp.float32),
        ],
    )(a_ud, a_lr, xf, wcat, wlin, ball)
    return h.reshape(num_sent, sent_len, out_dim)


# speedup vs baseline: 1.1766x; 1.1766x over previous
import functools

import jax
import jax.numpy as jnp
from jax.experimental import pallas as pl
from jax.experimental.pallas import tpu as pltpu


def _gnn_outer(nblocks, bm, n, hidden, out_dim,
               a_ud_hbm, a_lr_hbm, xf_hbm, xf_ref, wcat_ref, wlin_ref,
               ball_ref, out_hbm, y_ref):
    xf = xf_ref[...]
    y_ref[...] = jnp.dot(xf, wcat_ref[...], preferred_element_type=jnp.float32)

    def inner(a_ud_blk, a_lr_blk, xf_blk, out_blk):
        acc = jnp.dot(a_ud_blk[...], y_ref[:, :out_dim],
                      preferred_element_type=jnp.float32)
        acc = acc + jnp.dot(a_lr_blk[...], y_ref[:, out_dim:],
                            preferred_element_type=jnp.float32)
        acc = acc + (jnp.dot(xf_blk[...], wlin_ref[...],
                             preferred_element_type=jnp.float32)
                     + ball_ref[...])
        out_blk[...] = jnp.maximum(acc, 0.0)

    pipeline = pltpu.emit_pipeline(
        inner,
        grid=(nblocks,),
        in_specs=[
            pl.BlockSpec((bm, n), lambda i: (i, 0),
                         pipeline_mode=pl.Buffered(buffer_count=3)),
            pl.BlockSpec((bm, n), lambda i: (i, 0),
                         pipeline_mode=pl.Buffered(buffer_count=3)),
            pl.BlockSpec((bm, hidden), lambda i: (i, 0),
                         pipeline_mode=pl.Buffered(buffer_count=3)),
        ],
        out_specs=[pl.BlockSpec((bm, out_dim), lambda i: (i, 0))],
    )
    pipeline(a_ud_hbm, a_lr_hbm, xf_hbm, out_hbm)


def kernel(x, mask, a_ud, a_lr, W_lin, b_lin, W_ud, b_ud, W_lr, b_lr):
    num_sent, sent_len, hidden = x.shape
    n = num_sent * sent_len
    out_dim = W_lin.shape[0]
    xf = x.reshape(n, hidden)
    wcat = jnp.concatenate([W_ud.T, W_lr.T], axis=1)
    wlin = W_lin.T
    ball = (b_lin + b_ud + b_lr).reshape(1, out_dim)

    bm = 256
    nblocks = n // bm
    vmem = pltpu.MemorySpace.VMEM
    h = pl.pallas_call(
        functools.partial(_gnn_outer, nblocks, bm, n, hidden, out_dim),
        in_specs=[
            pl.BlockSpec(memory_space=pl.ANY),
            pl.BlockSpec(memory_space=pl.ANY),
            pl.BlockSpec(memory_space=pl.ANY),
            pl.BlockSpec(memory_space=vmem),
            pl.BlockSpec(memory_space=vmem),
            pl.BlockSpec(memory_space=vmem),
            pl.BlockSpec(memory_space=vmem),
        ],
        out_specs=pl.BlockSpec(memory_space=pl.ANY),
        out_shape=jax.ShapeDtypeStruct((n, out_dim), jnp.float32),
        scratch_shapes=[
            pltpu.VMEM((n, 2 * out_dim), jnp.float32),
        ],
    )(a_ud, a_lr, xf, xf, wcat, wlin, ball)
    return h.reshape(num_sent, sent_len, out_dim)


# speedup vs baseline: 1.2309x; 1.0462x over previous
import functools

import jax
import jax.numpy as jnp
from jax.experimental import pallas as pl
from jax.experimental.pallas import tpu as pltpu


def _gnn_block(out_dim, a_ud_ref, a_lr_ref, xf_ref, wcat_ref, wlin_ref,
               ball_ref, out_ref, y_ref, base_ref):
    i = pl.program_id(0)

    @pl.when(i == 0)
    def _():
        xf = xf_ref[...]
        y_ref[...] = jnp.dot(xf, wcat_ref[...],
                             preferred_element_type=jnp.float32)
        base_ref[...] = (jnp.dot(xf, wlin_ref[...],
                                 preferred_element_type=jnp.float32)
                         + ball_ref[...])

    y = y_ref[...]
    acc = jnp.dot(a_ud_ref[...], y[:, :out_dim],
                  preferred_element_type=jnp.float32)
    acc = acc + jnp.dot(a_lr_ref[...], y[:, out_dim:],
                        preferred_element_type=jnp.float32)
    bm = out_ref.shape[0]
    acc = acc + base_ref[pl.ds(i * bm, bm), :]
    out_ref[...] = jnp.maximum(acc, 0.0)


def kernel(x, mask, a_ud, a_lr, W_lin, b_lin, W_ud, b_ud, W_lr, b_lr):
    num_sent, sent_len, hidden = x.shape
    n = num_sent * sent_len
    out_dim = W_lin.shape[0]
    xf = x.reshape(n, hidden)
    wcat = jnp.concatenate([W_ud.T, W_lr.T], axis=1)
    wlin = W_lin.T
    ball = (b_lin + b_ud + b_lr).reshape(1, out_dim)

    bm = 256
    grid = (n // bm,)
    h = pl.pallas_call(
        functools.partial(_gnn_block, out_dim),
        grid=grid,
        in_specs=[
            pl.BlockSpec((bm, n), lambda i: (i, 0)),
            pl.BlockSpec((bm, n), lambda i: (i, 0)),
            pl.BlockSpec((n, hidden), lambda i: (0, 0)),
            pl.BlockSpec((hidden, 2 * out_dim), lambda i: (0, 0)),
            pl.BlockSpec((hidden, out_dim), lambda i: (0, 0)),
            pl.BlockSpec((1, out_dim), lambda i: (0, 0)),
        ],
        out_specs=pl.BlockSpec((bm, out_dim), lambda i: (i, 0)),
        out_shape=jax.ShapeDtypeStruct((n, out_dim), jnp.float32),
        scratch_shapes=[
            pltpu.VMEM((n, 2 * out_dim), jnp.float32),
            pltpu.VMEM((n, out_dim), jnp.float32),
        ],
    )(a_ud, a_lr, xf, wcat, wlin, ball)
    return h.reshape(num_sent, sent_len, out_dim)


# speedup vs baseline: 1.3683x; 1.1117x over previous
import functools

import jax
import jax.numpy as jnp
from jax.experimental import pallas as pl
from jax.experimental.pallas import tpu as pltpu


def _xwt(xf, w):
    return jax.lax.dot_general(xf, w, (((1,), (1,)), ((), ())),
                               preferred_element_type=jnp.float32)


def _gnn_block(out_dim, a_ud_ref, a_lr_ref, xf_ref, wlin_ref, wud_ref,
               wlr_ref, blin_ref, bud_ref, blr_ref, out_ref, y_ref, base_ref):
    i = pl.program_id(0)

    @pl.when(i == 0)
    def _():
        xf = xf_ref[...]
        y_ref[:, :out_dim] = _xwt(xf, wud_ref[...])
        y_ref[:, out_dim:] = _xwt(xf, wlr_ref[...])
        base_ref[...] = (_xwt(xf, wlin_ref[...])
                         + (blin_ref[...] + bud_ref[...] + blr_ref[...]))

    y = y_ref[...]
    acc = jnp.dot(a_ud_ref[...], y[:, :out_dim],
                  preferred_element_type=jnp.float32)
    acc = acc + jnp.dot(a_lr_ref[...], y[:, out_dim:],
                        preferred_element_type=jnp.float32)
    bm = out_ref.shape[0]
    acc = acc + base_ref[pl.ds(i * bm, bm), :]
    out_ref[...] = jnp.maximum(acc, 0.0)


def kernel(x, mask, a_ud, a_lr, W_lin, b_lin, W_ud, b_ud, W_lr, b_lr):
    num_sent, sent_len, hidden = x.shape
    n = num_sent * sent_len
    out_dim = W_lin.shape[0]
    xf = x.reshape(n, hidden)
    blin = b_lin.reshape(1, out_dim)
    bud = b_ud.reshape(1, out_dim)
    blr = b_lr.reshape(1, out_dim)

    bm = 256
    grid = (n // bm,)
    h = pl.pallas_call(
        functools.partial(_gnn_block, out_dim),
        grid=grid,
        in_specs=[
            pl.BlockSpec((bm, n), lambda i: (i, 0)),
            pl.BlockSpec((bm, n), lambda i: (i, 0)),
            pl.BlockSpec((n, hidden), lambda i: (0, 0)),
            pl.BlockSpec((out_dim, hidden), lambda i: (0, 0)),
            pl.BlockSpec((out_dim, hidden), lambda i: (0, 0)),
            pl.BlockSpec((out_dim, hidden), lambda i: (0, 0)),
            pl.BlockSpec((1, out_dim), lambda i: (0, 0)),
            pl.BlockSpec((1, out_dim), lambda i: (0, 0)),
            pl.BlockSpec((1, out_dim), lambda i: (0, 0)),
        ],
        out_specs=pl.BlockSpec((bm, out_dim), lambda i: (i, 0)),
        out_shape=jax.ShapeDtypeStruct((n, out_dim), jnp.float32),
        scratch_shapes=[
            pltpu.VMEM((n, 2 * out_dim), jnp.float32),
            pltpu.VMEM((n, out_dim), jnp.float32),
        ],
    )(a_ud, a_lr, xf, W_lin, W_ud, W_lr, blin, bud, blr)
    return h.reshape(num_sent, sent_len, out_dim)
